# baseline (speedup 1.0000x reference)
import functools

import jax
import jax.numpy as jnp
from jax import lax
from jax.experimental import pallas as pl
from jax.experimental.pallas import tpu as pltpu
from jax.experimental.pallas import tpu_sc as plsc

N, D = 16384, 512

SC_ROWS = 4096
TC_ROWS = N - SC_ROWS

NC, NS, L = 2, 16, 16
NW = NC * NS
ROWS_W = SC_ROWS // NW
CR = 32
NCHUNK = ROWS_W // CR
NACC = 8

_mesh = plsc.VectorSubcoreMesh(
    core_axis_name="c", subcore_axis_name="s", num_cores=NC, num_subcores=NS
)


@functools.partial(
    pl.kernel,
    out_type=jax.ShapeDtypeStruct((NW, L), jnp.float32),
    mesh=_mesh,
    scratch_types=[
        pltpu.VMEM((2, CR, D), jnp.float32),
        pltpu.VMEM((2, CR, D), jnp.float32),
        pltpu.VMEM((L,), jnp.float32),
        pltpu.SemaphoreType.DMA,
        pltpu.SemaphoreType.DMA,
    ],
    compiler_params=pltpu.CompilerParams(use_tc_tiling_on_sc=True),
)
def _sc_sse(pred_hbm, true_hbm, out_hbm, pbuf, tbuf, accv, sem0, sem1):
    wid = lax.axis_index("s") * NC + lax.axis_index("c")
    base = wid * ROWS_W
    sems = (sem0, sem1)

    def start(k):
        slot = k % 2
        row0 = base + k * CR
        cp = pltpu.async_copy(
            pred_hbm.at[pl.ds(row0, CR), :], pbuf.at[slot], sems[slot]
        )
        ct = pltpu.async_copy(
            true_hbm.at[pl.ds(row0, CR), :], tbuf.at[slot], sems[slot]
        )
        return cp, ct

    inflight = start(0)
    accs = tuple(jnp.zeros((L,), jnp.float32) for _ in range(NACC))
    for k in range(NCHUNK):
        slot = k % 2
        cp, ct = inflight
        cp.wait()
        ct.wait()
        if k + 1 < NCHUNK:
            inflight = start(k + 1)

        def body(r, a):
            a = list(a)
            for t in range(D // L):
                d = pbuf[slot, r, pl.ds(t * L, L)] - tbuf[slot, r, pl.ds(t * L, L)]
                a[t % NACC] = a[t % NACC] + d * d
            return tuple(a)

        accs = plsc.parallel_loop(0, CR, carry=accs)(body)

    acc = accs[0]
    for t in range(1, NACC):
        acc = acc + accs[t]
    accv[...] = acc
    pltpu.sync_copy(accv, out_hbm.at[wid])


BLOCK_ROWS = 2048
TC_GRID = TC_ROWS // BLOCK_ROWS
TC_BLOCK0 = SC_ROWS // BLOCK_ROWS


def _tc_sse(pred_ref, true_ref, out_ref, acc_ref):
    i = pl.program_id(0)

    @pl.when(i == 0)
    def _():
        acc_ref[...] = jnp.zeros_like(acc_ref)

    d = pred_ref[...] - true_ref[...]
    acc_ref[...] += jnp.sum(d * d, axis=0, keepdims=True)

    @pl.when(i == TC_GRID - 1)
    def _():
        out_ref[...] = jnp.sum(acc_ref[...]).reshape(1, 1)


def kernel(y_pred, y_true):
    sc_partials = _sc_sse(y_pred, y_true)
    d = y_pred[SC_ROWS:] - y_true[SC_ROWS:]
    tc_total = jnp.sum(d * d)
    return (jnp.sum(sc_partials) + tc_total) / N


# speedup vs baseline: 1.0068x; 1.0068x over previous
import functools

import jax
import jax.numpy as jnp
from jax import lax
from jax.experimental import pallas as pl
from jax.experimental.pallas import tpu as pltpu
from jax.experimental.pallas import tpu_sc as plsc

N, D = 16384, 512

SC_ROWS = 4096
TC_ROWS = N - SC_ROWS

NC, NS, L = 2, 16, 16
NW = NC * NS
ROWS_W = SC_ROWS // NW
CR = 32
NCHUNK = ROWS_W // CR
NACC = 8

_mesh = plsc.VectorSubcoreMesh(
    core_axis_name="c", subcore_axis_name="s", num_cores=NC, num_subcores=NS
)


@functools.partial(
    pl.kernel,
    out_type=jax.ShapeDtypeStruct((NW, L), jnp.float32),
    mesh=_mesh,
    scratch_types=[
        pltpu.VMEM((2, CR, D), jnp.float32),
        pltpu.VMEM((2, CR, D), jnp.float32),
        pltpu.VMEM((L,), jnp.float32),
        pltpu.SemaphoreType.DMA,
        pltpu.SemaphoreType.DMA,
    ],
    compiler_params=pltpu.CompilerParams(use_tc_tiling_on_sc=True),
    cost_estimate=pl.CostEstimate(
        flops=3 * SC_ROWS * D,
        bytes_accessed=2 * 4 * SC_ROWS * D,
        transcendentals=0,
    ),
)
def _sc_sse(pred_hbm, true_hbm, out_hbm, pbuf, tbuf, accv, sem0, sem1):
    wid = lax.axis_index("s") * NC + lax.axis_index("c")
    base = wid * ROWS_W
    sems = (sem0, sem1)

    def start(k):
        slot = k % 2
        row0 = base + k * CR
        cp = pltpu.async_copy(
            pred_hbm.at[pl.ds(row0, CR), :], pbuf.at[slot], sems[slot]
        )
        ct = pltpu.async_copy(
            true_hbm.at[pl.ds(row0, CR), :], tbuf.at[slot], sems[slot]
        )
        return cp, ct

    inflight = start(0)
    accs = tuple(jnp.zeros((L,), jnp.float32) for _ in range(NACC))
    for k in range(NCHUNK):
        slot = k % 2
        cp, ct = inflight
        cp.wait()
        ct.wait()
        if k + 1 < NCHUNK:
            inflight = start(k + 1)

        def body(r, a):
            a = list(a)
            for t in range(D // L):
                d = pbuf[slot, r, pl.ds(t * L, L)] - tbuf[slot, r, pl.ds(t * L, L)]
                a[t % NACC] = a[t % NACC] + d * d
            return tuple(a)

        accs = plsc.parallel_loop(0, CR, carry=accs)(body)

    acc = accs[0]
    for t in range(1, NACC):
        acc = acc + accs[t]
    accv[...] = acc
    pltpu.sync_copy(accv, out_hbm.at[wid])


BLOCK_ROWS = 2048
TC_GRID = TC_ROWS // BLOCK_ROWS
TC_BLOCK0 = SC_ROWS // BLOCK_ROWS


def _tc_sse(pred_ref, true_ref, out_ref, acc_ref):
    i = pl.program_id(0)

    @pl.when(i == 0)
    def _():
        acc_ref[...] = jnp.zeros_like(acc_ref)

    d = pred_ref[...] - true_ref[...]
    acc_ref[...] += jnp.sum(d * d, axis=0, keepdims=True)

    @pl.when(i == TC_GRID - 1)
    def _():
        out_ref[...] = jnp.sum(acc_ref[...]).reshape(1, 1)


def kernel(y_pred, y_true):
    tc_total = pl.pallas_call(
        _tc_sse,
        grid=(TC_GRID,),
        in_specs=[
            pl.BlockSpec((BLOCK_ROWS, D), lambda i: (i + TC_BLOCK0, 0)),
            pl.BlockSpec((BLOCK_ROWS, D), lambda i: (i + TC_BLOCK0, 0)),
        ],
        out_specs=pl.BlockSpec((1, 1), lambda i: (0, 0)),
        out_shape=jax.ShapeDtypeStruct((1, 1), jnp.float32),
        scratch_shapes=[pltpu.VMEM((1, D), jnp.float32)],
    )(y_pred, y_true)
    sc_partials = _sc_sse(y_pred, y_true)
    return (jnp.sum(sc_partials) + tc_total[0, 0]) / N


# speedup vs baseline: 1.0082x; 1.0013x over previous
import functools

import jax
import jax.numpy as jnp
from jax import lax
from jax.experimental import pallas as pl
from jax.experimental.pallas import tpu as pltpu
from jax.experimental.pallas import tpu_sc as plsc

N, D = 16384, 512

SC_ROWS = 4096
TC_ROWS = N - SC_ROWS

NC, NS, L = 2, 16, 16
NW = NC * NS
ROWS_W = SC_ROWS // NW
CR = 32
NCHUNK = ROWS_W // CR
NACC = 8

_mesh = plsc.VectorSubcoreMesh(
    core_axis_name="c", subcore_axis_name="s", num_cores=NC, num_subcores=NS
)


@functools.partial(
    pl.kernel,
    out_type=jax.ShapeDtypeStruct((NW, L), jnp.float32),
    mesh=_mesh,
    scratch_types=[
        pltpu.VMEM((2, CR, D), jnp.float32),
        pltpu.VMEM((2, CR, D), jnp.float32),
        pltpu.VMEM((L,), jnp.float32),
        pltpu.SemaphoreType.DMA,
        pltpu.SemaphoreType.DMA,
    ],
    compiler_params=pltpu.CompilerParams(use_tc_tiling_on_sc=True),
    cost_estimate=pl.CostEstimate(
        flops=3 * SC_ROWS * D,
        bytes_accessed=2 * 4 * SC_ROWS * D,
        transcendentals=0,
    ),
)
def _sc_sse(pred_hbm, true_hbm, out_hbm, pbuf, tbuf, accv, sem0, sem1):
    wid = lax.axis_index("s") * NC + lax.axis_index("c")
    base = wid * ROWS_W
    sems = (sem0, sem1)

    def start(k):
        slot = k % 2
        row0 = base + k * CR
        cp = pltpu.async_copy(
            pred_hbm.at[pl.ds(row0, CR), :], pbuf.at[slot], sems[slot]
        )
        ct = pltpu.async_copy(
            true_hbm.at[pl.ds(row0, CR), :], tbuf.at[slot], sems[slot]
        )
        return cp, ct

    inflight = start(0)
    accs = tuple(jnp.zeros((L,), jnp.float32) for _ in range(NACC))
    for k in range(NCHUNK):
        slot = k % 2
        cp, ct = inflight
        cp.wait()
        ct.wait()
        if k + 1 < NCHUNK:
            inflight = start(k + 1)

        def body(r, a):
            a = list(a)
            for t in range(D // L):
                d = pbuf[slot, r, pl.ds(t * L, L)] - tbuf[slot, r, pl.ds(t * L, L)]
                a[t % NACC] = a[t % NACC] + d * d
            return tuple(a)

        accs = plsc.parallel_loop(0, CR, carry=accs)(body)

    acc = accs[0]
    for t in range(1, NACC):
        acc = acc + accs[t]
    accv[...] = acc
    pltpu.sync_copy(accv, out_hbm.at[wid])


BLOCK_ROWS = 2048
TC_GRID = TC_ROWS // BLOCK_ROWS
TC_BLOCK0 = SC_ROWS // BLOCK_ROWS


def _tc_sse(pred_ref, true_ref, out_ref, acc_ref):
    i = pl.program_id(0)

    @pl.when(i == 0)
    def _():
        acc_ref[...] = jnp.zeros_like(acc_ref)

    d = pred_ref[...] - true_ref[...]
    acc_ref[...] += jnp.sum(d * d, axis=0, keepdims=True)

    @pl.when(i == TC_GRID - 1)
    def _():
        out_ref[...] = jnp.sum(acc_ref[...]).reshape(1, 1)


def kernel(y_pred, y_true):
    tc_total = pl.pallas_call(
        _tc_sse,
        grid=(TC_GRID,),
        in_specs=[
            pl.BlockSpec((BLOCK_ROWS, D), lambda i: (i + TC_BLOCK0, 0)),
            pl.BlockSpec((BLOCK_ROWS, D), lambda i: (i + TC_BLOCK0, 0)),
        ],
        out_specs=pl.BlockSpec((1, 1), lambda i: (0, 0)),
        out_shape=jax.ShapeDtypeStruct((1, 1), jnp.float32),
        scratch_shapes=[pltpu.VMEM((1, D), jnp.float32)],
        compiler_params=pltpu.CompilerParams(skip_device_barrier=True),
    )(y_pred, y_true)
    sc_partials = _sc_sse(y_pred, y_true)
    return (jnp.sum(sc_partials) + tc_total[0, 0]) / N


# speedup vs baseline: 1.8032x; 1.7886x over previous
import jax
import jax.numpy as jnp
from jax import lax
from jax.experimental import pallas as pl
from jax.experimental.pallas import tpu as pltpu

N, D = 16384, 512
CHR = 512
NCHUNK = N // CHR
NBUF = 4


def _sse_stream(pred_hbm, true_hbm, out_ref, pbuf, tbuf, acc_ref, sems):
    def start(k, slot):
        pltpu.make_async_copy(
            pred_hbm.at[pl.ds(k * CHR, CHR), :],
            pbuf.at[slot],
            sems.at[slot, 0],
        ).start()
        pltpu.make_async_copy(
            true_hbm.at[pl.ds(k * CHR, CHR), :],
            tbuf.at[slot],
            sems.at[slot, 1],
        ).start()

    for k in range(NBUF):
        start(k, k)

    acc_ref[...] = jnp.zeros_like(acc_ref)

    def body(k, _):
        slot = lax.rem(k, NBUF)
        pltpu.make_async_copy(
            pred_hbm.at[pl.ds(0, CHR), :], pbuf.at[slot], sems.at[slot, 0]
        ).wait()
        pltpu.make_async_copy(
            true_hbm.at[pl.ds(0, CHR), :], tbuf.at[slot], sems.at[slot, 1]
        ).wait()
        d = pbuf[slot] - tbuf[slot]
        acc_ref[...] += jnp.sum(d * d, axis=0, keepdims=True)

        @pl.when(k + NBUF < NCHUNK)
        def _():
            nk = k + NBUF

            def dyn_start(hbm, buf, s):
                pltpu.make_async_copy(
                    hbm.at[pl.ds(nk * CHR, CHR), :],
                    buf.at[slot],
                    sems.at[slot, s],
                ).start()

            dyn_start(pred_hbm, pbuf, 0)
            dyn_start(true_hbm, tbuf, 1)

        return 0

    lax.fori_loop(0, NCHUNK, body, 0)
    out_ref[...] = jnp.sum(acc_ref[...]).reshape(1, 1)


def kernel(y_pred, y_true):
    total = pl.pallas_call(
        _sse_stream,
        in_specs=[
            pl.BlockSpec(memory_space=pl.ANY),
            pl.BlockSpec(memory_space=pl.ANY),
        ],
        out_specs=pl.BlockSpec(memory_space=pltpu.MemorySpace.VMEM),
        out_shape=jax.ShapeDtypeStruct((1, 1), jnp.float32),
        scratch_shapes=[
            pltpu.VMEM((NBUF, CHR, D), jnp.float32),
            pltpu.VMEM((NBUF, CHR, D), jnp.float32),
            pltpu.VMEM((1, D), jnp.float32),
            pltpu.SemaphoreType.DMA((NBUF, 2)),
        ],
    )(y_pred, y_true)
    return total[0, 0] / N


# speedup vs baseline: 1.8390x; 1.0199x over previous
import jax
import jax.numpy as jnp
from jax import lax
from jax.experimental import pallas as pl
from jax.experimental.pallas import tpu as pltpu

N, D = 16384, 512
CHR = 256
NCHUNK = N // CHR
NBUF = 8


def _sse_stream(pred_hbm, true_hbm, out_ref, pbuf, tbuf, acc_ref, sems):
    def start(k, slot):
        pltpu.make_async_copy(
            pred_hbm.at[pl.ds(k * CHR, CHR), :],
            pbuf.at[slot],
            sems.at[slot, 0],
        ).start()
        pltpu.make_async_copy(
            true_hbm.at[pl.ds(k * CHR, CHR), :],
            tbuf.at[slot],
            sems.at[slot, 1],
        ).start()

    for k in range(NBUF):
        start(k, k)

    acc_ref[...] = jnp.zeros_like(acc_ref)

    def body(k, _):
        slot = lax.rem(k, NBUF)
        pltpu.make_async_copy(
            pred_hbm.at[pl.ds(0, CHR), :], pbuf.at[slot], sems.at[slot, 0]
        ).wait()
        pltpu.make_async_copy(
            true_hbm.at[pl.ds(0, CHR), :], tbuf.at[slot], sems.at[slot, 1]
        ).wait()
        d = pbuf[slot] - tbuf[slot]
        acc_ref[...] += jnp.sum(d * d, axis=0, keepdims=True)

        @pl.when(k + NBUF < NCHUNK)
        def _():
            nk = k + NBUF

            def dyn_start(hbm, buf, s):
                pltpu.make_async_copy(
                    hbm.at[pl.ds(nk * CHR, CHR), :],
                    buf.at[slot],
                    sems.at[slot, s],
                ).start()

            dyn_start(pred_hbm, pbuf, 0)
            dyn_start(true_hbm, tbuf, 1)

        return 0

    lax.fori_loop(0, NCHUNK, body, 0)
    out_ref[...] = jnp.sum(acc_ref[...]).reshape(1, 1)


def kernel(y_pred, y_true):
    total = pl.pallas_call(
        _sse_stream,
        in_specs=[
            pl.BlockSpec(memory_space=pl.ANY),
            pl.BlockSpec(memory_space=pl.ANY),
        ],
        out_specs=pl.BlockSpec(memory_space=pltpu.MemorySpace.VMEM),
        out_shape=jax.ShapeDtypeStruct((1, 1), jnp.float32),
        scratch_shapes=[
            pltpu.VMEM((NBUF, CHR, D), jnp.float32),
            pltpu.VMEM((NBUF, CHR, D), jnp.float32),
            pltpu.VMEM((1, D), jnp.float32),
            pltpu.SemaphoreType.DMA((NBUF, 2)),
        ],
    )(y_pred, y_true)
    return total[0, 0] / N
